# Initial kernel scaffold; baseline (speedup 1.0000x reference)
import jax
import jax.numpy as jnp
from jax.experimental import pallas as pl


def kernel(x, attention_mask, Wq, bq, Wk, bk, Wv, bv, Wo, bo, g1, be1, g2, be2, Wsw, bsw, W_fc1, b_fc1, W_fc2, b_fc2):
    raise NotImplementedError("write your pallas kernel here")


# speedup vs baseline: 1.0054x; 1.0054x over previous
import functools

import jax
import jax.numpy as jnp
from jax import lax
from jax.experimental import pallas as pl
from jax.experimental.pallas import tpu as pltpu
from jax.experimental.pallas import tpu_sc as plsc

B, S, D, H = 2, 2048, 1024, 16
DH = D // H
E = 8
FF = 4096
EPS = 1e-12
CAP = (B * S) // E
N = B * S
RB = 256
NRB = N // RB

f32 = jnp.float32
i32 = jnp.int32


def _ln(x, g, b):
    u = x.mean(-1, keepdims=True)
    s = ((x - u) ** 2).mean(-1, keepdims=True)
    return g * ((x - u) / jnp.sqrt(s + EPS)) + b


def _ln_qkv_body(x_ref, g_ref, b_ref, wq_ref, bq_ref, wk_ref, bk_ref,
                 wv_ref, bv_ref, q_ref, k_ref, v_ref):
    xn = _ln(x_ref[...], g_ref[...], b_ref[...])
    qb = jnp.dot(xn, wq_ref[...], preferred_element_type=f32) + bq_ref[...]
    kb = jnp.dot(xn, wk_ref[...], preferred_element_type=f32) + bk_ref[...]
    vb = jnp.dot(xn, wv_ref[...], preferred_element_type=f32) + bv_ref[...]
    for h in range(H):
        q_ref[h] = qb[:, h * DH:(h + 1) * DH]
        k_ref[h] = kb[:, h * DH:(h + 1) * DH]
        v_ref[h] = vb[:, h * DH:(h + 1) * DH]


def _ln_qkv(xf, g1, be1, Wq, bq, Wk, bk, Wv, bv):
    row = pl.BlockSpec((RB, D), lambda i: (i, 0))
    vec = pl.BlockSpec((1, D), lambda i: (0, 0))
    mat = pl.BlockSpec((D, D), lambda i: (0, 0))
    hspec = pl.BlockSpec((H, RB, DH), lambda i: (0, i, 0))
    out = jax.ShapeDtypeStruct((H, N, DH), f32)
    return pl.pallas_call(
        _ln_qkv_body,
        grid=(NRB,),
        in_specs=[row, vec, vec, mat, vec, mat, vec, mat, vec],
        out_specs=[hspec, hspec, hspec],
        out_shape=[out, out, out],
    )(xf, g1.reshape(1, D), be1.reshape(1, D), Wq, bq.reshape(1, D),
      Wk, bk.reshape(1, D), Wv, bv.reshape(1, D))


def _attn_body(q_ref, k_ref, v_ref, m_ref, scores_ref, ctx_ref):
    s = lax.dot_general(q_ref[0], k_ref[0], (((1,), (1,)), ((), ())),
                        preferred_element_type=f32) * 0.125
    s = s + m_ref[0]
    scores_ref[...] = s.reshape(1, 1, RB, S)
    mx = jnp.max(s, axis=-1, keepdims=True)
    p = jnp.exp(s - mx)
    p = p / jnp.sum(p, axis=-1, keepdims=True)
    ctx_ref[0] = jnp.dot(p, v_ref[0], preferred_element_type=f32)


def _attn(q, k, v, mask2):
    qspec = pl.BlockSpec((1, RB, DH), lambda b, h, c: (h, b * (S // RB) + c, 0))
    kvspec = pl.BlockSpec((1, S, DH), lambda b, h, c: (h, b, 0))
    mspec = pl.BlockSpec((1, 1, S), lambda b, h, c: (b, 0, 0))
    return pl.pallas_call(
        _attn_body,
        grid=(B, H, S // RB),
        in_specs=[qspec, kvspec, kvspec, mspec],
        out_specs=[pl.BlockSpec((1, 1, RB, S), lambda b, h, c: (b, h, c, 0)),
                   qspec],
        out_shape=[jax.ShapeDtypeStruct((B, H, S, S), f32),
                   jax.ShapeDtypeStruct((H, N, DH), f32)],
    )(q, k, v, mask2)


def _post_body(x_ref, ctx_ref, wo_ref, bo_ref, g_ref, b_ref, wsw_ref, bsw_ref,
               x1_ref, xn2_ref, routes_ref):
    ctx = jnp.concatenate([ctx_ref[h] for h in range(H)], axis=-1)
    x1 = x_ref[...] + jnp.dot(ctx, wo_ref[...],
                              preferred_element_type=f32) + bo_ref[...]
    x1_ref[...] = x1
    xn2 = _ln(x1, g_ref[...], b_ref[...])
    xn2_ref[...] = xn2
    logits = jnp.dot(xn2, wsw_ref[...], preferred_element_type=f32) + bsw_ref[...]
    mx = jnp.max(logits, axis=-1, keepdims=True)
    ids = lax.broadcasted_iota(i32, (RB, E), 1)
    routes = jnp.min(jnp.where(logits == mx, ids, E), axis=-1, keepdims=True)
    routes_ref[...] = routes.reshape(1, RB, 1)


def _post(xf, ctx, Wo, bo, g2, be2, Wsw, bsw):
    row = pl.BlockSpec((RB, D), lambda i: (i, 0))
    vec = pl.BlockSpec((1, D), lambda i: (0, 0))
    return pl.pallas_call(
        _post_body,
        grid=(NRB,),
        in_specs=[row, pl.BlockSpec((H, RB, DH), lambda i: (0, i, 0)),
                  pl.BlockSpec((D, D), lambda i: (0, 0)), vec, vec, vec,
                  pl.BlockSpec((D, E), lambda i: (0, 0)),
                  pl.BlockSpec((1, E), lambda i: (0, 0))],
        out_specs=[row, row, pl.BlockSpec((1, RB, 1), lambda i: (i, 0, 0))],
        out_shape=[jax.ShapeDtypeStruct((N, D), f32),
                   jax.ShapeDtypeStruct((N, D), f32),
                   jax.ShapeDtypeStruct((NRB, RB, 1), i32)],
    )(xf, ctx, Wo, bo.reshape(1, D), g2.reshape(1, D), be2.reshape(1, D),
      Wsw, bsw.reshape(1, E))


def _positions_body(r_ref, cidx_ref, valid_ref, sdst_ref):
    r = r_ref[...]
    il = lax.broadcasted_iota(i32, (128, 128), 0)
    jl = lax.broadcasted_iota(i32, (128, 128), 1)
    M = (il <= jl).astype(f32)
    ir = lax.broadcasted_iota(i32, (32, 32), 0)
    jr = lax.broadcasted_iota(i32, (32, 32), 1)
    Ls = (jr < ir).astype(f32)
    pos = jnp.zeros((32, 128), i32)
    for e in range(E):
        ohe = (r == e).astype(f32)
        incl = jnp.dot(ohe, M, preferred_element_type=f32)
        rowtot = jnp.sum(ohe, axis=1, keepdims=True)
        off = jnp.dot(Ls, rowtot, preferred_element_type=f32)
        cnt = (incl + off).astype(i32)
        pos = pos + jnp.where(r == e, cnt - 1, 0)
    valid = (pos < CAP).astype(i32)
    dst = r * CAP + pos
    valid_ref[...] = valid
    sdst_ref[...] = jnp.where(valid == 1, dst, 0)
    cidx_ref[...] = r * CAP + jnp.minimum(pos, CAP - 1)


def _positions(routes2d):
    spec = pl.BlockSpec((32, 128), lambda: (0, 0))
    o = jax.ShapeDtypeStruct((32, 128), i32)
    return pl.pallas_call(
        _positions_body,
        grid=(),
        in_specs=[spec],
        out_specs=[spec, spec, spec],
        out_shape=[o, o, o],
    )(routes2d)


def _sc_mesh():
    return plsc.VectorSubcoreMesh(core_axis_name="c", subcore_axis_name="s")


_RPT = N // 32
_CH = 64


def _sc_dispatch(xs, sdst, valid):
    @functools.partial(
        pl.kernel,
        out_type=jax.ShapeDtypeStruct((N, D), f32),
        mesh=_sc_mesh(),
        scratch_types=[
            pltpu.VMEM((N,), i32),
            pltpu.VMEM((N,), i32),
            pltpu.VMEM((N,), i32),
            pltpu.VMEM((_CH, D), f32),
            pltpu.SemaphoreType.DMA,
        ],
        compiler_params=pltpu.CompilerParams(needs_layout_passes=False),
    )
    def k(xs_hbm, sdst_hbm, valid_hbm, out_hbm, sdst_v, valid_v, slot_v,
          rows_v, sem):
        pltpu.sync_copy(sdst_hbm, sdst_v)
        pltpu.sync_copy(valid_hbm, valid_v)

        def init(i, _):
            slot_v[pl.ds(i * 16, 16)] = jnp.zeros((16,), i32)
            return 0
        lax.fori_loop(0, N // 16, init, 0)

        def scat(i, _):
            d = sdst_v[pl.ds(i * 16, 16)]
            m = valid_v[pl.ds(i * 16, 16)] == 1
            toks = lax.iota(i32, 16) + i * 16
            plsc.store_scatter(slot_v, [d], toks, mask=m)
            return 0
        lax.fori_loop(0, N // 16, scat, 0)

        wid = lax.axis_index("s") * 2 + lax.axis_index("c")
        base = wid * _RPT
        for ch in range(_RPT // _CH):
            off = base + ch * _CH
            pltpu.async_copy(xs_hbm.at[slot_v.at[pl.ds(off, _CH)]],
                             rows_v, sem).wait()
            pltpu.sync_copy(rows_v, out_hbm.at[pl.ds(off, _CH)])

    return k(xs, sdst, valid)


def _sc_combine(table, cidx):
    @functools.partial(
        pl.kernel,
        out_type=jax.ShapeDtypeStruct((N, D), f32),
        mesh=_sc_mesh(),
        scratch_types=[
            pltpu.VMEM((_RPT,), i32),
            pltpu.VMEM((_CH, D), f32),
            pltpu.SemaphoreType.DMA,
        ],
    )
    def k(table_hbm, cidx_hbm, out_hbm, idx_v, rows_v, sem):
        wid = lax.axis_index("s") * 2 + lax.axis_index("c")
        base = wid * _RPT
        pltpu.sync_copy(cidx_hbm.at[pl.ds(base, _RPT)], idx_v)
        for ch in range(_RPT // _CH):
            pltpu.async_copy(table_hbm.at[idx_v.at[pl.ds(ch * _CH, _CH)]],
                             rows_v, sem).wait()
            pltpu.sync_copy(rows_v, out_hbm.at[pl.ds(base + ch * _CH, _CH)])

    return k(table, cidx)


_FC = 1024


def _ffn_body(in_ref, w1_ref, b1_ref, w2_ref, b2_ref, out_ref, acc_ref):
    fi = pl.program_id(1)
    h = jnp.dot(in_ref[0], w1_ref[0], preferred_element_type=f32) + b1_ref[0, 0, 0]
    h = h * 0.5 * (1.0 + lax.erf(h / 1.41421))
    contrib = jnp.dot(h, w2_ref[0], preferred_element_type=f32)

    @pl.when(fi == 0)
    def _():
        acc_ref[...] = contrib

    @pl.when(fi > 0)
    def _():
        acc_ref[...] = acc_ref[...] + contrib

    @pl.when(fi == FF // _FC - 1)
    def _():
        out_ref[0] = acc_ref[...] + b2_ref[0]


def _ffn(expert_in, W_fc1, b_fc1, W_fc2, b_fc2):
    return pl.pallas_call(
        _ffn_body,
        grid=(E, FF // _FC),
        in_specs=[pl.BlockSpec((1, CAP, D), lambda e, fi: (e, 0, 0)),
                  pl.BlockSpec((1, D, _FC), lambda e, fi: (e, 0, fi)),
                  pl.BlockSpec((1, 1, 1, _FC), lambda e, fi: (e, fi, 0, 0)),
                  pl.BlockSpec((1, _FC, D), lambda e, fi: (e, fi, 0)),
                  pl.BlockSpec((1, 1, D), lambda e, fi: (e, 0, 0))],
        out_specs=pl.BlockSpec((1, CAP, D), lambda e, fi: (e, 0, 0)),
        out_shape=jax.ShapeDtypeStruct((E, CAP, D), f32),
        scratch_shapes=[pltpu.VMEM((CAP, D), f32)],
    )(expert_in, W_fc1, b_fc1.reshape(E, FF // _FC, 1, _FC),
      W_fc2, b_fc2.reshape(E, 1, D))


def _final_body(g_ref, xs_ref, x1_ref, v_ref, out_ref):
    sel = jnp.where(v_ref[...] == 1, g_ref[...], xs_ref[...])
    out_ref[...] = sel + x1_ref[...]


def _final(gathered, xs, x1, validc):
    row = pl.BlockSpec((RB, D), lambda i: (i, 0))
    return pl.pallas_call(
        _final_body,
        grid=(NRB,),
        in_specs=[row, row, row, pl.BlockSpec((RB, 1), lambda i: (i, 0))],
        out_specs=row,
        out_shape=jax.ShapeDtypeStruct((N, D), f32),
    )(gathered, xs, x1, validc)


def kernel(x, attention_mask, Wq, bq, Wk, bk, Wv, bv, Wo, bo, g1, be1,
           g2, be2, Wsw, bsw, W_fc1, b_fc1, W_fc2, b_fc2):
    xf = x.reshape(N, D)
    mask2 = attention_mask.reshape(B, 1, S)
    q, k, v = _ln_qkv(xf, g1, be1, Wq, bq, Wk, bk, Wv, bv)
    scores, ctx = _attn(q, k, v, mask2)
    x1, xn2, routes3 = _post(xf, ctx, Wo, bo, g2, be2, Wsw, bsw)
    routes2d = routes3.reshape(32, 128)
    cidx, validm, sdst = _positions(routes2d)
    expert_in = _sc_dispatch(xn2, sdst.reshape(N), validm.reshape(N))
    expert_out = _ffn(expert_in.reshape(E, CAP, D), W_fc1, b_fc1, W_fc2, b_fc2)
    gathered = _sc_combine(expert_out.reshape(N, D), cidx.reshape(N))
    x2 = _final(gathered, xn2, x1, validm.reshape(N, 1)).reshape(B, S, D)
    return x2, scores
